# baseline (speedup 1.0000x reference)
import functools

import jax
import jax.numpy as jnp
from jax import lax
from jax.experimental import pallas as pl
from jax.experimental.pallas import tpu as pltpu
from jax.experimental.pallas import tpu_sc as plsc

D_MODEL = 128
LANES = 16
NUM_CORES = 2
NUM_SUBCORES = 16
NUM_WORKERS = NUM_CORES * NUM_SUBCORES
CHUNK = 128
NBUF = 4
UNROLL = 8


@jax.jit
def _run(x1d, idx2d, pe1d):
    n_words = x1d.shape[0]
    n = n_words // D_MODEL
    rows_per_w = n // NUM_WORKERS
    chunks = rows_per_w // CHUNK
    mesh = plsc.VectorSubcoreMesh(core_axis_name="c", subcore_axis_name="s")

    @functools.partial(
        pl.kernel,
        mesh=mesh,
        compiler_params=pltpu.CompilerParams(needs_layout_passes=False),
        out_type=jax.ShapeDtypeStruct((n_words,), jnp.float32),
        scratch_types=[
            pltpu.VMEM((pe1d.shape[0],), jnp.float32),
            pltpu.VMEM((chunks, CHUNK), jnp.int32),
        ]
        + [pltpu.VMEM((CHUNK * D_MODEL,), jnp.float32) for _ in range(NBUF)]
        + [pltpu.SemaphoreType.DMA for _ in range(2 * NBUF)],
    )
    def k(x_hbm, idx_hbm, pe_hbm, out_hbm, pe_v, idxs, *rest):
        xb = rest[:NBUF]
        sin = rest[NBUF:2 * NBUF]
        sout = rest[2 * NBUF:3 * NBUF]
        wid = lax.axis_index("s") * NUM_CORES + lax.axis_index("c")

        pltpu.sync_copy(pe_hbm, pe_v)
        pltpu.sync_copy(idx_hbm.at[pl.ds(wid * chunks, chunks)], idxs)

        def in_copy(g, b):
            base = (wid * rows_per_w + g * CHUNK) * D_MODEL
            return pltpu.make_async_copy(
                x_hbm.at[pl.ds(base, CHUNK * D_MODEL)], xb[b], sin[b])

        def out_copy(g, b):
            base = (wid * rows_per_w + g * CHUNK) * D_MODEL
            return pltpu.make_async_copy(
                xb[b], out_hbm.at[pl.ds(base, CHUNK * D_MODEL)], sout[b])

        in_copy(0, 0).start()
        in_copy(1, 1).start()

        lane = lax.iota(jnp.int32, LANES)
        xoff = lane * D_MODEL

        def compute(g, b):
            buf = xb[b]

            def row_group(rg, carry):
                rv = idxs[g, pl.ds(rg * LANES, LANES)]
                pe_base = rv * D_MODEL
                x_base = rg * (LANES * D_MODEL) + xoff

                def cols(t, c2):
                    for u in range(UNROLL):
                        c = t * UNROLL + u
                        pa = plsc.load_gather(pe_v, [pe_base + c])
                        xv = plsc.load_gather(buf, [x_base + c])
                        plsc.store_scatter(buf, [x_base + c], xv + pa)
                    return c2

                return lax.fori_loop(0, D_MODEL // UNROLL, cols, carry)

            lax.fori_loop(0, CHUNK // LANES, row_group, 0)

        def step(g, b):
            b2 = (b + 2) % NBUF

            @pl.when(g + 2 < chunks)
            def _():
                @pl.when(g + 2 >= NBUF)
                def _():
                    out_copy(g + 2 - NBUF, b2).wait()

                in_copy(g + 2, b2).start()

            in_copy(g, b).wait()
            compute(g, b)
            out_copy(g, b).start()

        def block(h, carry):
            for u in range(NBUF):
                step(NBUF * h + u, u)
            return carry

        lax.fori_loop(0, chunks // NBUF, block, 0)
        for g in range(chunks - NBUF, chunks):
            out_copy(g, g % NBUF).wait()

    return k(x1d, idx2d, pe1d)


def kernel(x, layer_index, pe):
    pe1d = pe.reshape(-1)
    idx2d = layer_index.reshape(layer_index.shape[0] // CHUNK, CHUNK)
    out = _run(x.reshape(-1), idx2d, pe1d)
    return out.reshape(x.shape)


# speedup vs baseline: 12.8233x; 12.8233x over previous
import functools

import jax
import jax.numpy as jnp
from jax import lax
from jax.experimental import pallas as pl
from jax.experimental.pallas import tpu as pltpu
from jax.experimental.pallas import tpu_sc as plsc

D_MODEL = 128
NUM_CORES = 2
NUM_SUBCORES = 16
NUM_WORKERS = NUM_CORES * NUM_SUBCORES
CHUNK = 128
NBUF = 4


@jax.jit
def _run(x, idx2d, pe2d):
    n = x.shape[0]
    rows_per_w = n // NUM_WORKERS
    chunks = rows_per_w // CHUNK
    assert chunks % NBUF == 0 and chunks >= 2 * NBUF
    mesh = plsc.VectorSubcoreMesh(core_axis_name="c", subcore_axis_name="s")

    @functools.partial(
        pl.kernel,
        mesh=mesh,
        out_type=jax.ShapeDtypeStruct((n, D_MODEL), jnp.float32),
        scratch_types=[
            pltpu.VMEM_SHARED((100, D_MODEL), jnp.float32),
            pltpu.VMEM_SHARED(
                (2, NUM_SUBCORES, CHUNK, D_MODEL), jnp.float32
            ),
            pltpu.VMEM((chunks, CHUNK), jnp.int32),
        ]
        + [pltpu.VMEM((CHUNK, D_MODEL), jnp.float32) for _ in range(NBUF)]
        + [pltpu.SemaphoreType.DMA for _ in range(4 * NBUF)],
    )
    def k(x_hbm, idx_hbm, pe_hbm, out_hbm, pe_sh, stage_sh, idxs, *rest):
        xb = rest[:NBUF]
        sin = rest[NBUF:2 * NBUF]
        sadd = rest[2 * NBUF:3 * NBUF]
        sst = rest[3 * NBUF:4 * NBUF]
        sho = rest[4 * NBUF:5 * NBUF]
        sid = lax.axis_index("s")
        wid = sid * NUM_CORES + lax.axis_index("c")

        @pl.when(sid == 0)
        def _():
            pltpu.sync_copy(pe_hbm, pe_sh)

        pltpu.sync_copy(idx_hbm.at[pl.ds(wid * chunks, chunks)], idxs)
        plsc.subcore_barrier()

        def in_copy(g, b):
            base = wid * rows_per_w + g * CHUNK
            return pltpu.make_async_copy(
                x_hbm.at[pl.ds(base, CHUNK)], xb[b], sin[b])

        def add_copy(g, b):
            return pltpu.make_async_copy(pe_sh.at[idxs.at[g]], xb[b], sadd[b])

        def stage_copy(g, b):
            return pltpu.make_async_copy(
                xb[b], stage_sh.at[b % 2, sid], sst[b])

        def hbm_copy(g, b):
            base = wid * rows_per_w + g * CHUNK
            return pltpu.make_async_copy(
                stage_sh.at[b % 2, sid], out_hbm.at[pl.ds(base, CHUNK)],
                sho[b])

        in_copy(0, 0).start()
        in_copy(1, 1).start()

        def step(g, b):
            b2 = (b + 2) % NBUF
            b1 = (b - 1) % NBUF

            @pl.when(g >= 2)
            def _():
                stage_copy(g - 2, b2).wait()
                hbm_copy(g - 2, b2).start()

            @pl.when(g + 2 < chunks)
            def _():
                in_copy(g + 2, b2).start()

            in_copy(g, b).wait()
            add_copy(g, b).start(add=True)

            @pl.when(g >= 1)
            def _():
                add_copy(g - 1, b1).wait()

                @pl.when(g >= 4)
                def _():
                    hbm_copy(g - 3, (b - 3) % NBUF).wait()

                stage_copy(g - 1, b1).start()

        def block(h, carry):
            for u in range(NBUF):
                step(NBUF * h + u, u)
            return carry

        lax.fori_loop(0, chunks // NBUF, block, 0)

        last = chunks - 1
        hbm_copy(0, 0).wait()
        add_copy(last, last % NBUF).wait()
        hbm_copy(last - 2, (last - 2) % NBUF).wait()
        stage_copy(last, last % NBUF).start()
        for g in (chunks - 2, chunks - 1):
            stage_copy(g, g % NBUF).wait()
            hbm_copy(g, g % NBUF).start()
        for g in (chunks - 2, chunks - 1):
            hbm_copy(g, g % NBUF).wait()

    return k(x, idx2d, pe2d)


def kernel(x, layer_index, pe):
    pe2d = pe.reshape(pe.shape[0], pe.shape[-1])
    idx2d = layer_index.reshape(layer_index.shape[0] // CHUNK, CHUNK)
    return _run(x, idx2d, pe2d)


# speedup vs baseline: 16.5977x; 1.2943x over previous
import functools

import jax
import jax.numpy as jnp
from jax import lax
from jax.experimental import pallas as pl
from jax.experimental.pallas import tpu as pltpu
from jax.experimental.pallas import tpu_sc as plsc

D_MODEL = 128
NUM_CORES = 2
NUM_SUBCORES = 16
NUM_WORKERS = NUM_CORES * NUM_SUBCORES
CHUNK = 128
NBUF = 4


@jax.jit
def _run(x, idx2d, pe2d):
    n = x.shape[0]
    rows_per_w = n // NUM_WORKERS
    chunks = rows_per_w // CHUNK
    assert chunks % NBUF == 0 and NBUF >= 3
    mesh = plsc.VectorSubcoreMesh(core_axis_name="c", subcore_axis_name="s")

    @functools.partial(
        pl.kernel,
        mesh=mesh,
        out_type=jax.ShapeDtypeStruct((n, D_MODEL), jnp.float32),
        scratch_types=[
            pltpu.VMEM_SHARED((100, D_MODEL), jnp.float32),
            pltpu.VMEM((chunks, CHUNK), jnp.int32),
        ]
        + [pltpu.VMEM((CHUNK, D_MODEL), jnp.float32) for _ in range(NBUF)]
        + [pltpu.SemaphoreType.DMA for _ in range(3 * NBUF)],
    )
    def k(x_hbm, idx_hbm, pe_hbm, out_hbm, pe_sh, idxs, *rest):
        xb = rest[:NBUF]
        sin = rest[NBUF:2 * NBUF]
        sadd = rest[2 * NBUF:3 * NBUF]
        sout = rest[3 * NBUF:4 * NBUF]
        wid = lax.axis_index("s") * NUM_CORES + lax.axis_index("c")

        @pl.when(lax.axis_index("s") == 0)
        def _():
            pltpu.sync_copy(pe_hbm, pe_sh)

        pltpu.sync_copy(idx_hbm.at[pl.ds(wid * chunks, chunks)], idxs)
        plsc.subcore_barrier()

        def in_copy(g, b):
            base = wid * rows_per_w + g * CHUNK
            return pltpu.make_async_copy(
                x_hbm.at[pl.ds(base, CHUNK)], xb[b], sin[b])

        def add_copy(g, b):
            return pltpu.make_async_copy(pe_sh.at[idxs.at[g]], xb[b], sadd[b])

        def out_copy(g, b):
            base = wid * rows_per_w + g * CHUNK
            return pltpu.make_async_copy(
                xb[b], out_hbm.at[pl.ds(base, CHUNK)], sout[b])

        in_copy(0, 0).start()
        in_copy(1, 1).start()

        def step(g, b):
            b2 = (b + 2) % NBUF
            b1 = (b - 1) % NBUF

            @pl.when(g + 2 < chunks)
            def _():
                @pl.when(g + 2 >= NBUF)
                def _():
                    out_copy(g + 2 - NBUF, b2).wait()

                in_copy(g + 2, b2).start()

            in_copy(g, b).wait()
            add_copy(g, b).start(add=True)

            @pl.when(g >= 1)
            def _():
                add_copy(g - 1, b1).wait()
                out_copy(g - 1, b1).start()

        def block(h, carry):
            for u in range(NBUF):
                step(NBUF * h + u, u)
            return carry

        lax.fori_loop(0, chunks // NBUF, block, 0)
        last = chunks - 1
        add_copy(last, last % NBUF).wait()
        out_copy(last, last % NBUF).start()
        for g in range(chunks - NBUF, chunks):
            out_copy(g, g % NBUF).wait()

    return k(x, idx2d, pe2d)


def kernel(x, layer_index, pe):
    pe2d = pe.reshape(pe.shape[0], pe.shape[-1])
    idx2d = layer_index.reshape(layer_index.shape[0] // CHUNK, CHUNK)
    return _run(x, idx2d, pe2d)
